# baseline (speedup 1.0000x reference)
import jax
import jax.numpy as jnp
from jax.experimental import pallas as pl
from jax.experimental.pallas import tpu as pltpu

P = 512
EMB = 32
H = 64
BP = 16

_DN0 = (((0,), (0,)), ((), ()))


def _lstm_block(corrT_ref, htT_ref, ctT_ref, nei_ref,
                wembT_ref, bembT_ref, wz_ref,
                houtT_ref, coutT_ref):
    wemb = wembT_ref[...].T
    wx = wemb[:, 0:1]
    wy = wemb[:, 1:2]
    bembT = bembT_ref[...]
    wz = wz_ref[...]
    ones = jnp.ones((1, P), jnp.float32)

    for j in range(BP):
        htj = htT_ref[j]
        ctj = ctT_ref[j]
        cx = corrT_ref[j, 0:1, :]
        cy = corrT_ref[j, 1:2, :]
        m = nei_ref[j:j + 1, :] > 0

        embT = jnp.maximum(wx * cx + wy * cy + bembT, 0.0)
        z = jnp.concatenate([embT, htj, ones], axis=0)
        gates = jax.lax.dot_general(wz, z, _DN0,
                                    preferred_element_type=jnp.float32)

        t_if = jnp.tanh(0.5 * gates[0 * H:2 * H])
        i_g = 0.5 + 0.5 * t_if[0 * H:1 * H]
        f_g = 0.5 + 0.5 * t_if[1 * H:2 * H]
        g_g = jnp.tanh(gates[2 * H:3 * H])
        o_g = 0.5 + 0.5 * jnp.tanh(0.5 * gates[3 * H:4 * H])

        c_new = f_g * ctj + i_g * g_g
        h_new = o_g * jnp.tanh(c_new)

        houtT_ref[j] = jnp.where(m, h_new, htj)
        coutT_ref[j] = jnp.where(m, c_new, ctj)


def kernel(corr_index, rela_ht, rela_ct, nei_index, W_emb, b_emb, W_ih, b_ih, W_hh, b_hh):
    corrT = corr_index.transpose(0, 2, 1)
    htT = rela_ht.transpose(0, 2, 1)
    ctT = rela_ct.transpose(0, 2, 1)

    wembT = W_emb.T
    bembT = b_emb.reshape(EMB, 1)
    wz = jnp.concatenate(
        [W_ih.T, W_hh.T, (b_ih + b_hh).reshape(1, 4 * H)], axis=0)

    grid = (P // BP,)
    spec_corr = pl.BlockSpec((BP, 2, P), lambda i: (i, 0, 0))
    spec_state = pl.BlockSpec((BP, H, P), lambda i: (i, 0, 0))
    spec_nei = pl.BlockSpec((BP, P), lambda i: (i, 0))
    full = lambda a: pl.BlockSpec(a.shape, lambda i: (0, 0))

    houtT, coutT = pl.pallas_call(
        _lstm_block,
        grid=grid,
        in_specs=[
            spec_corr, spec_state, spec_state, spec_nei,
            full(wembT), full(bembT), full(wz),
        ],
        out_specs=[spec_state, spec_state],
        out_shape=[
            jax.ShapeDtypeStruct((P, H, P), jnp.float32),
            jax.ShapeDtypeStruct((P, H, P), jnp.float32),
        ],
        compiler_params=pltpu.CompilerParams(
            dimension_semantics=("parallel",),
        ),
    )(corrT, htT, ctT, nei_index, wembT, bembT, wz)

    return (houtT.transpose(0, 2, 1), coutT.transpose(0, 2, 1))


# speedup vs baseline: 1.0746x; 1.0746x over previous
import jax
import jax.numpy as jnp
from jax.experimental import pallas as pl
from jax.experimental.pallas import tpu as pltpu

P = 512
EMB = 32
H = 64
BP = 32

_DN0 = (((0,), (0,)), ((), ()))


def _lstm_block(corrT_ref, htT_ref, ctT_ref, nei_ref,
                wembT_ref, bembT_ref, wz_ref,
                houtT_ref, coutT_ref):
    wemb = wembT_ref[...].T
    wx = wemb[:, 0:1]
    wy = wemb[:, 1:2]
    bembT = bembT_ref[...]
    wz = wz_ref[...]
    ones = jnp.ones((1, P), jnp.float32)

    for j in range(BP):
        htj = htT_ref[j]
        ctj = ctT_ref[j]
        cx = corrT_ref[j, 0:1, :]
        cy = corrT_ref[j, 1:2, :]
        m = nei_ref[j:j + 1, :] > 0

        embT = jnp.maximum(wx * cx + wy * cy + bembT, 0.0)
        z = jnp.concatenate([embT, htj, ones], axis=0)
        gates = jax.lax.dot_general(wz, z, _DN0,
                                    preferred_element_type=jnp.float32)

        t_if = jnp.tanh(0.5 * gates[0 * H:2 * H])
        i_g = 0.5 + 0.5 * t_if[0 * H:1 * H]
        f_g = 0.5 + 0.5 * t_if[1 * H:2 * H]
        g_g = jnp.tanh(gates[2 * H:3 * H])
        o_g = 0.5 + 0.5 * jnp.tanh(0.5 * gates[3 * H:4 * H])

        c_new = f_g * ctj + i_g * g_g
        h_new = o_g * jnp.tanh(c_new)

        houtT_ref[j] = jnp.where(m, h_new, htj)
        coutT_ref[j] = jnp.where(m, c_new, ctj)


def kernel(corr_index, rela_ht, rela_ct, nei_index, W_emb, b_emb, W_ih, b_ih, W_hh, b_hh):
    corrT = corr_index.transpose(0, 2, 1)
    htT = rela_ht.transpose(0, 2, 1)
    ctT = rela_ct.transpose(0, 2, 1)

    wembT = W_emb.T
    bembT = b_emb.reshape(EMB, 1)
    wz = jnp.concatenate(
        [W_ih.T, W_hh.T, (b_ih + b_hh).reshape(1, 4 * H)], axis=0)

    grid = (P // BP,)
    spec_corr = pl.BlockSpec((BP, 2, P), lambda i: (i, 0, 0))
    spec_state = pl.BlockSpec((BP, H, P), lambda i: (i, 0, 0))
    spec_nei = pl.BlockSpec((BP, P), lambda i: (i, 0))
    full = lambda a: pl.BlockSpec(a.shape, lambda i: (0, 0))

    houtT, coutT = pl.pallas_call(
        _lstm_block,
        grid=grid,
        in_specs=[
            spec_corr, spec_state, spec_state, spec_nei,
            full(wembT), full(bembT), full(wz),
        ],
        out_specs=[spec_state, spec_state],
        out_shape=[
            jax.ShapeDtypeStruct((P, H, P), jnp.float32),
            jax.ShapeDtypeStruct((P, H, P), jnp.float32),
        ],
        compiler_params=pltpu.CompilerParams(
            dimension_semantics=("parallel",),
        ),
    )(corrT, htT, ctT, nei_index, wembT, bembT, wz)

    return (houtT.transpose(0, 2, 1), coutT.transpose(0, 2, 1))
